# baseline (speedup 1.0000x reference)
import functools

import jax
import jax.numpy as jnp
from jax import lax
from jax.experimental import pallas as pl
from jax.experimental.pallas import tpu as pltpu
from jax.experimental.pallas import tpu_sc as plsc

_NC = 2
_NS = 16


def kernel(x, W):
    B, P, D = x.shape
    nb = B // _NS
    Dh = D // _NC
    Dq = Dh // 2
    Wt = jnp.swapaxes(W, 0, 1)
    mesh = plsc.VectorSubcoreMesh(core_axis_name="c", subcore_axis_name="s")

    @functools.partial(
        pl.kernel,
        out_type=jax.ShapeDtypeStruct((B, D, P), W.dtype),
        mesh=mesh,
        scratch_types=[
            pltpu.VMEM((Dh, P), W.dtype),
            pltpu.SemaphoreType.DMA,
            pltpu.SemaphoreType.DMA,
        ],
    )
    def sc_broadcast(w_hbm, out_hbm, wbuf, sem_in, sem_out):
        c = lax.axis_index("c")
        s = lax.axis_index("s")
        base = s * nb
        d0 = c * Dh
        stage = [
            pltpu.make_async_copy(
                w_hbm.at[pl.ds(d0 + q * Dq, Dq)],
                wbuf.at[pl.ds(q * Dq, Dq)],
                sem_in,
            )
            for q in range(2)
        ]
        for st in stage:
            st.start()
        out_copies = []
        for q in range(2):
            stage[q].wait()
            copies = [
                pltpu.make_async_copy(
                    wbuf.at[pl.ds(q * Dq, Dq)],
                    out_hbm.at[base + i, pl.ds(d0 + q * Dq, Dq)],
                    sem_out,
                )
                for i in range(nb)
            ]
            for cp in copies:
                cp.start()
            out_copies += copies
        for cp in out_copies:
            cp.wait()

    out_t = sc_broadcast(Wt)
    return jnp.swapaxes(out_t, 1, 2)


# speedup vs baseline: 1.0402x; 1.0402x over previous
import functools

import jax
import jax.numpy as jnp
from jax import lax
from jax.experimental import pallas as pl
from jax.experimental.pallas import tpu as pltpu
from jax.experimental.pallas import tpu_sc as plsc

_NC = 2
_NS = 16


def kernel(x, W):
    B, P, D = x.shape
    nw = _NC * _NS
    ng = nw // 2
    Dh = D // 2
    ns = 3
    Bs = ns * nw
    Bt = B - Bs
    nb = Bt // ng
    Wt = jnp.swapaxes(W, 0, 1)
    mesh = plsc.VectorSubcoreMesh(core_axis_name="c", subcore_axis_name="s")

    @functools.partial(
        pl.kernel,
        out_type=jax.ShapeDtypeStruct((B, D, P), W.dtype),
        mesh=mesh,
        scratch_types=[
            pltpu.VMEM((Dh, P), W.dtype),
            pltpu.VMEM_SHARED((D, P), W.dtype),
            pltpu.SemaphoreType.DMA,
            pltpu.SemaphoreType.DMA,
        ],
    )
    def sc_broadcast(w_hbm, out_hbm, wbuf, wshared, sem_stage, sem_out):
        c = lax.axis_index("c")
        s = lax.axis_index("s")
        wid = c * _NS + s
        half = wid % 2
        group = wid // 2

        shared_stage = pltpu.make_async_copy(w_hbm, wshared, sem_stage)

        @pl.when(s == 0)
        def _start_shared_stage():
            shared_stage.start()

        pltpu.sync_copy(w_hbm.at[pl.ds(half * Dh, Dh)], wbuf)
        base = group * nb
        tile_copies = [
            pltpu.make_async_copy(
                wbuf, out_hbm.at[base + i, pl.ds(half * Dh, Dh)], sem_out
            )
            for i in range(nb)
        ]
        for cp in tile_copies:
            cp.start()

        @pl.when(s == 0)
        def _finish_shared_stage():
            shared_stage.wait()

        plsc.subcore_barrier()

        sbase = Bt + wid * ns
        shared_copies = [
            pltpu.make_async_copy(wshared, out_hbm.at[sbase + i], sem_out)
            for i in range(ns)
        ]
        for cp in shared_copies:
            cp.start()
        for cp in tile_copies + shared_copies:
            cp.wait()

    out_t = sc_broadcast(Wt)
    return jnp.swapaxes(out_t, 1, 2)


# speedup vs baseline: 1.0416x; 1.0013x over previous
import functools

import jax
import jax.numpy as jnp
from jax import lax
from jax.experimental import pallas as pl
from jax.experimental.pallas import tpu as pltpu
from jax.experimental.pallas import tpu_sc as plsc

_NC = 2
_NS = 16


def kernel(x, W):
    B, P, D = x.shape
    nw = _NC * _NS
    ng = nw // 2
    nb = B // ng
    Dh = D // 2
    Wt = jnp.swapaxes(W, 0, 1)
    mesh = plsc.VectorSubcoreMesh(core_axis_name="c", subcore_axis_name="s")

    @functools.partial(
        pl.kernel,
        out_type=jax.ShapeDtypeStruct((B, D, P), W.dtype),
        mesh=mesh,
        scratch_types=[
            pltpu.VMEM((Dh, P), W.dtype),
            pltpu.SemaphoreType.DMA,
        ],
    )
    def sc_broadcast(w_hbm, out_hbm, wbuf, sem):
        c = lax.axis_index("c")
        s = lax.axis_index("s")
        wid = c * _NS + s
        half = wid % 2
        group = wid // 2
        pltpu.sync_copy(w_hbm.at[pl.ds(half * Dh, Dh)], wbuf)
        base = group * nb
        copies = [
            pltpu.make_async_copy(
                wbuf, out_hbm.at[base + i, pl.ds(half * Dh, Dh)], sem
            )
            for i in range(nb)
        ]
        for cp in copies:
            cp.start()
        for cp in copies:
            cp.wait()

    out_t = sc_broadcast(Wt)
    return jnp.swapaxes(out_t, 1, 2)
